# Initial kernel scaffold; baseline (speedup 1.0000x reference)
import jax
import jax.numpy as jnp
from jax.experimental import pallas as pl


def kernel(sample, neg_idx, entity_table, rel_table):
    raise NotImplementedError("write your pallas kernel here")


# speedup vs baseline: 2.9663x; 2.9663x over previous
import functools

import jax
import jax.numpy as jnp
from jax import lax
from jax.experimental import pallas as pl
from jax.experimental.pallas import tpu as pltpu
from jax.experimental.pallas import tpu_sc as plsc

B = 4096
D = 128
NEG = 8
NC = 2
NS = 16
NW = NC * NS
BPW = B // NW
NCH = BPW // 16
NOFF = 3 * B
EPS = 1e-8
MARGIN = 1.0


def _sc_body(ent_hbm, rel_hbm, idx_hbm,
             negsq_hbm, pos32_hbm,
             hidx_v, ridx_v, tidx_v, nidx_v,
             hbuf, rbuf, tbuf, nbuf0, nbuf1,
             possq_v, negsq_v,
             sem_h, sem_r, sem_t, sem_n0, sem_n1):
    wid = lax.axis_index("s") * NC + lax.axis_index("c")
    base = wid * BPW
    nbase = NOFF + wid * (BPW * NEG)

    pltpu.sync_copy(idx_hbm.at[pl.ds(base, BPW)], hidx_v)
    pltpu.sync_copy(idx_hbm.at[pl.ds(B + base, BPW)], ridx_v)
    pltpu.sync_copy(idx_hbm.at[pl.ds(2 * B + base, BPW)], tidx_v)
    pltpu.sync_copy(idx_hbm.at[pl.ds(nbase, BPW * NEG)], nidx_v)

    cp_h = pltpu.async_copy(ent_hbm.at[hidx_v], hbuf, sem_h)
    cp_r = pltpu.async_copy(rel_hbm.at[ridx_v], rbuf, sem_r)
    cp_t = pltpu.async_copy(ent_hbm.at[tidx_v], tbuf, sem_t)
    nbufs = (nbuf0, nbuf1)
    nsems = (sem_n0, sem_n1)
    pending = [
        pltpu.async_copy(ent_hbm.at[nidx_v.at[pl.ds(0, 128)]], nbuf0, sem_n0),
        pltpu.async_copy(ent_hbm.at[nidx_v.at[pl.ds(128, 128)]], nbuf1, sem_n1),
    ]
    cp_h.wait()
    cp_r.wait()
    cp_t.wait()

    iota = lax.broadcasted_iota(jnp.int32, (16,), 0)
    perms = [jnp.bitwise_and(iota + sh, 15) for sh in (8, 4, 2, 1)]

    def _lanesum(acc):
        for perm in perms:
            acc = acc + jnp.take(acc, perm)
        return acc

    for j in range(NCH):
        nbuf = nbufs[j % 2]
        pending[j].wait()

        def per_b(m, carry, j=j, nbuf=nbuf):
            bl = j * 16 + m
            lane_m = iota == m
            teps = [tbuf[bl, pl.ds(c * 16, 16)] + EPS for c in range(8)]
            acc = jnp.zeros((16,), jnp.float32)
            for c in range(8):
                dv = (teps[c] - hbuf[bl, pl.ds(c * 16, 16)]
                      - rbuf[bl, pl.ds(c * 16, 16)])
                acc = acc + dv * dv
            res = list(carry)
            res[0] = jnp.where(lane_m, _lanesum(acc), res[0])
            for k in range(NEG):
                rr = m * NEG + k
                acc2 = jnp.zeros((16,), jnp.float32)
                for c in range(8):
                    dv = teps[c] - nbuf[rr, pl.ds(c * 16, 16)]
                    acc2 = acc2 + dv * dv
                res[1 + k] = jnp.where(lane_m, _lanesum(acc2), res[1 + k])
            return tuple(res)

        zero16 = jnp.zeros((16,), jnp.float32)
        out_vecs = lax.fori_loop(0, 16, per_b, (zero16,) * (1 + NEG))
        possq_v[pl.ds(j * 16, 16)] = out_vecs[0]
        for k in range(NEG):
            negsq_v[k, pl.ds(j * 16, 16)] = out_vecs[1 + k]
        if j + 2 < NCH:
            pending.append(pltpu.async_copy(
                ent_hbm.at[nidx_v.at[pl.ds((j + 2) * 128, 128)]],
                nbufs[j % 2], nsems[j % 2]))

    for k in range(NEG):
        pltpu.sync_copy(negsq_v.at[k], negsq_hbm.at[k, pl.ds(base, BPW)])
    pltpu.sync_copy(possq_v, pos32_hbm.at[wid])


@functools.partial(
    pl.kernel,
    out_type=(jax.ShapeDtypeStruct((NEG, B), jnp.float32),
              jax.ShapeDtypeStruct((NW, BPW), jnp.float32)),
    mesh=plsc.VectorSubcoreMesh(core_axis_name="c", subcore_axis_name="s"),
    scratch_types=[
        pltpu.VMEM((BPW,), jnp.int32),
        pltpu.VMEM((BPW,), jnp.int32),
        pltpu.VMEM((BPW,), jnp.int32),
        pltpu.VMEM((BPW * NEG,), jnp.int32),
        pltpu.VMEM((BPW, D), jnp.float32),
        pltpu.VMEM((BPW, D), jnp.float32),
        pltpu.VMEM((BPW, D), jnp.float32),
        pltpu.VMEM((128, D), jnp.float32),
        pltpu.VMEM((128, D), jnp.float32),
        pltpu.VMEM((BPW,), jnp.float32),
        pltpu.VMEM((NEG, BPW), jnp.float32),
        pltpu.SemaphoreType.DMA,
        pltpu.SemaphoreType.DMA,
        pltpu.SemaphoreType.DMA,
        pltpu.SemaphoreType.DMA,
        pltpu.SemaphoreType.DMA,
    ],
)
def _sc_call(ent_hbm, rel_hbm, idx_hbm, negsq_hbm, pos32_hbm, *scratch):
    _sc_body(ent_hbm, rel_hbm, idx_hbm, negsq_hbm, pos32_hbm, *scratch)


def _tc_body(negsq_ref, pos32_ref, out_ref):
    pos = jnp.sqrt(pos32_ref[:, :])
    negd = jnp.sqrt(negsq_ref[:, :])
    neg = jnp.mean(negd, axis=0, keepdims=True)
    pos_sum = jnp.sum(pos)
    neg_sum = jnp.sum(neg)
    out_ref[:, :] = jnp.broadcast_to((pos_sum - neg_sum) / B + MARGIN, (1, 1))

    pos_min = jnp.min(pos)
    neg_max = jnp.max(neg)

    @pl.when(pos_min + MARGIN < neg_max)
    def _brute():
        at = jnp.transpose(pos)

        tot = jnp.float32(0.0)
        for ci in range(NW):
            a = at[:, ci:ci + 1]
            blk = jnp.maximum(a - neg + MARGIN, 0.0)
            tot = tot + jnp.sum(blk)
        out_ref[:, :] = jnp.broadcast_to(tot / (B * B), (1, 1))


def _tc_call(negsq, pos32):
    return pl.pallas_call(
        _tc_body,
        out_shape=jax.ShapeDtypeStruct((1, 1), jnp.float32),
    )(negsq, pos32)


def kernel(sample, neg_idx, entity_table, rel_table):
    s32 = sample.astype(jnp.int32)
    idx_all = jnp.concatenate(
        [s32[:, 0], s32[:, 1], s32[:, 2],
         neg_idx.astype(jnp.int32).reshape(B * NEG)])
    negsq, pos32 = _sc_call(entity_table, rel_table, idx_all)
    total = _tc_call(negsq, pos32)
    return total[0, 0]


# speedup vs baseline: 3.1653x; 1.0671x over previous
import functools

import jax
import jax.numpy as jnp
from jax import lax
from jax.experimental import pallas as pl
from jax.experimental.pallas import tpu as pltpu
from jax.experimental.pallas import tpu_sc as plsc

B = 4096
D = 128
NEG = 8
NC = 2
NS = 16
NW = NC * NS
BPW = B // NW
NCH = BPW // 16
NOFF = 3 * B
EPS = 1e-8
MARGIN = 1.0


def _sc_body(ent_hbm, rel_hbm, idx_hbm,
             negsq_hbm, pos32_hbm,
             hidx_v, ridx_v, tidx_v, nidx_v,
             hbuf, rbuf, tbuf, nbuf0, nbuf1,
             possq_v, negsq8_v,
             sem_h, sem_r, sem_t, sem_n0, sem_n1):
    wid = lax.axis_index("s") * NC + lax.axis_index("c")
    base = wid * BPW
    nbase = NOFF + wid * (BPW * NEG)

    pltpu.sync_copy(idx_hbm.at[pl.ds(base, BPW)], hidx_v)
    pltpu.sync_copy(idx_hbm.at[pl.ds(B + base, BPW)], ridx_v)
    pltpu.sync_copy(idx_hbm.at[pl.ds(2 * B + base, BPW)], tidx_v)
    pltpu.sync_copy(idx_hbm.at[pl.ds(nbase, BPW * NEG)], nidx_v)

    cp_h = pltpu.async_copy(ent_hbm.at[hidx_v], hbuf, sem_h)
    cp_r = pltpu.async_copy(rel_hbm.at[ridx_v], rbuf, sem_r)
    cp_t = pltpu.async_copy(ent_hbm.at[tidx_v], tbuf, sem_t)
    nbufs = (nbuf0, nbuf1)
    nsems = (sem_n0, sem_n1)
    pending = [
        pltpu.async_copy(ent_hbm.at[nidx_v.at[pl.ds(0, 128)]], nbuf0, sem_n0),
        pltpu.async_copy(ent_hbm.at[nidx_v.at[pl.ds(128, 128)]], nbuf1, sem_n1),
    ]
    cp_h.wait()
    cp_r.wait()
    cp_t.wait()

    iota = lax.broadcasted_iota(jnp.int32, (16,), 0)
    perms = [jnp.bitwise_and(iota + sh, 15) for sh in (8, 4, 2, 1)]
    x8 = jnp.bitwise_xor(iota, 8)
    x4 = jnp.bitwise_xor(iota, 4)
    x2 = jnp.bitwise_xor(iota, 2)
    x1 = jnp.bitwise_xor(iota, 1)
    m8 = jnp.bitwise_and(iota, 8) == 0
    m4 = jnp.bitwise_and(iota, 4) == 0
    m2 = jnp.bitwise_and(iota, 2) == 0
    lo8 = iota < 8
    peven = jnp.bitwise_and(iota * 2, 15)

    def _lanesum(acc):
        for perm in perms:
            acc = acc + jnp.take(acc, perm)
        return acc

    def _butterfly8(a):
        b = [jnp.where(m8, a[k], a[k + 4])
             + jnp.where(m8, jnp.take(a[k], x8), jnp.take(a[k + 4], x8))
             for k in range(4)]
        c = [jnp.where(m4, b[k], b[k + 2])
             + jnp.where(m4, jnp.take(b[k], x4), jnp.take(b[k + 2], x4))
             for k in range(2)]
        d = (jnp.where(m2, c[0], c[1])
             + jnp.where(m2, jnp.take(c[0], x2), jnp.take(c[1], x2)))
        return d + jnp.take(d, x1)

    for j in range(NCH):
        nbuf = nbufs[j % 2]
        pending[j].wait()

        def row_sums(m, nbuf=nbuf, j=j):
            bl = j * 16 + m
            teps = [tbuf[bl, pl.ds(c * 16, 16)] + EPS for c in range(8)]
            acc = jnp.zeros((16,), jnp.float32)
            for c in range(8):
                dv = (teps[c] - hbuf[bl, pl.ds(c * 16, 16)]
                      - rbuf[bl, pl.ds(c * 16, 16)])
                acc = acc + dv * dv
            accs = []
            for k in range(NEG):
                rr = m * NEG + k
                acc2 = jnp.zeros((16,), jnp.float32)
                for c in range(8):
                    dv = teps[c] - nbuf[rr, pl.ds(c * 16, 16)]
                    acc2 = acc2 + dv * dv
                accs.append(acc2)
            return _lanesum(acc), jnp.take(_butterfly8(accs), peven)

        def per_pair(i, pos_pack, j=j):
            m0 = 2 * i
            p0, w0 = row_sums(m0)
            p1, w1 = row_sums(m0 + 1)
            pos_pack = jnp.where(iota == m0, p0, pos_pack)
            pos_pack = jnp.where(iota == m0 + 1, p1, pos_pack)
            negsq8_v[pl.ds(j * 128 + i * 16, 16)] = jnp.where(lo8, w0, w1)
            return pos_pack

        zero16 = jnp.zeros((16,), jnp.float32)
        pos_pack = lax.fori_loop(0, 8, per_pair, zero16)
        possq_v[pl.ds(j * 16, 16)] = pos_pack
        if j + 2 < NCH:
            pending.append(pltpu.async_copy(
                ent_hbm.at[nidx_v.at[pl.ds((j + 2) * 128, 128)]],
                nbufs[j % 2], nsems[j % 2]))

    pltpu.sync_copy(negsq8_v, negsq_hbm.at[pl.ds(base * NEG, BPW * NEG)])
    pltpu.sync_copy(possq_v, pos32_hbm.at[wid])


@functools.partial(
    pl.kernel,
    out_type=(jax.ShapeDtypeStruct((B * NEG,), jnp.float32),
              jax.ShapeDtypeStruct((NW, BPW), jnp.float32)),
    mesh=plsc.VectorSubcoreMesh(core_axis_name="c", subcore_axis_name="s"),
    scratch_types=[
        pltpu.VMEM((BPW,), jnp.int32),
        pltpu.VMEM((BPW,), jnp.int32),
        pltpu.VMEM((BPW,), jnp.int32),
        pltpu.VMEM((BPW * NEG,), jnp.int32),
        pltpu.VMEM((BPW, D), jnp.float32),
        pltpu.VMEM((BPW, D), jnp.float32),
        pltpu.VMEM((BPW, D), jnp.float32),
        pltpu.VMEM((128, D), jnp.float32),
        pltpu.VMEM((128, D), jnp.float32),
        pltpu.VMEM((BPW,), jnp.float32),
        pltpu.VMEM((BPW * NEG,), jnp.float32),
        pltpu.SemaphoreType.DMA,
        pltpu.SemaphoreType.DMA,
        pltpu.SemaphoreType.DMA,
        pltpu.SemaphoreType.DMA,
        pltpu.SemaphoreType.DMA,
    ],
)
def _sc_call(ent_hbm, rel_hbm, idx_hbm, negsq_hbm, pos32_hbm, *scratch):
    _sc_body(ent_hbm, rel_hbm, idx_hbm, negsq_hbm, pos32_hbm, *scratch)


def _tc_body(negsq_ref, pos32_ref, out_ref):
    pos = jnp.sqrt(pos32_ref[:, :])
    negd = jnp.sqrt(negsq_ref[:, :])
    pos_sum = jnp.sum(pos)
    neg_sum = jnp.sum(negd) / NEG
    out_ref[:, :] = jnp.broadcast_to((pos_sum - neg_sum) / B + MARGIN, (1, 1))

    pos_min = jnp.min(pos)
    negd_max = jnp.max(negd)

    @pl.when(pos_min + MARGIN < negd_max)
    def _brute():
        li = lax.broadcasted_iota(jnp.int32, (D, 16), 0) // NEG
        gi = lax.broadcasted_iota(jnp.int32, (D, 16), 1)
        sel = jnp.where(li == gi, jnp.float32(1.0 / NEG), jnp.float32(0.0))
        nm16 = jax.lax.dot(negd, sel,
                           precision=jax.lax.Precision.HIGHEST)

        acc = jnp.zeros((256, D), jnp.float32)
        for rp in range(NW):
            prow = pos[rp:rp + 1, :]
            for g in range(16):
                a = nm16[:, g:g + 1]
                acc = acc + jnp.maximum(prow - a + MARGIN, 0.0)
        out_ref[:, :] = jnp.broadcast_to(jnp.sum(acc) / (B * B), (1, 1))


def _tc_call(negsq, pos32):
    return pl.pallas_call(
        _tc_body,
        out_shape=jax.ShapeDtypeStruct((1, 1), jnp.float32),
    )(negsq, pos32)


def kernel(sample, neg_idx, entity_table, rel_table):
    s32 = sample.astype(jnp.int32)
    idx_all = jnp.concatenate(
        [s32[:, 0], s32[:, 1], s32[:, 2],
         neg_idx.astype(jnp.int32).reshape(B * NEG)])
    negsq, pos32 = _sc_call(entity_table, rel_table, idx_all)
    total = _tc_call(negsq.reshape(B * NEG // D, D), pos32)
    return total[0, 0]


# speedup vs baseline: 3.2020x; 1.0116x over previous
import functools

import jax
import jax.numpy as jnp
from jax import lax
from jax.experimental import pallas as pl
from jax.experimental.pallas import tpu as pltpu
from jax.experimental.pallas import tpu_sc as plsc

B = 4096
D = 128
NEG = 8
NC = 2
NS = 16
NW = NC * NS
BPW = B // NW
NCH = BPW // 16
NOFF = 3 * B
EPS = 1e-8
MARGIN = 1.0


def _sc_body(ent_hbm, rel_hbm, idx_hbm,
             negsq_hbm, pos32_hbm,
             hidx_v, ridx_v, tidx_v, nidx_v,
             hbuf, rbuf, tbuf, nbuf0, nbuf1, nbuf2, nbuf3,
             possq_v, negsq8_v,
             sem_h, sem_r, sem_t, sem_n0, sem_n1, sem_n2, sem_n3):
    wid = lax.axis_index("s") * NC + lax.axis_index("c")
    base = wid * BPW
    nbase = NOFF + wid * (BPW * NEG)

    pltpu.sync_copy(idx_hbm.at[pl.ds(base, BPW)], hidx_v)
    pltpu.sync_copy(idx_hbm.at[pl.ds(B + base, BPW)], ridx_v)
    pltpu.sync_copy(idx_hbm.at[pl.ds(2 * B + base, BPW)], tidx_v)
    pltpu.sync_copy(idx_hbm.at[pl.ds(nbase, BPW * NEG)], nidx_v)

    cp_h = pltpu.async_copy(ent_hbm.at[hidx_v], hbuf, sem_h)
    cp_r = pltpu.async_copy(rel_hbm.at[ridx_v], rbuf, sem_r)
    cp_t = pltpu.async_copy(ent_hbm.at[tidx_v], tbuf, sem_t)
    nbufs = (nbuf0, nbuf1, nbuf2, nbuf3)
    nsems = (sem_n0, sem_n1, sem_n2, sem_n3)
    pending = [
        pltpu.async_copy(ent_hbm.at[nidx_v.at[pl.ds(k * 128, 128)]],
                         nbufs[k], nsems[k])
        for k in range(4)
    ]
    cp_h.wait()
    cp_r.wait()
    cp_t.wait()

    iota = lax.broadcasted_iota(jnp.int32, (16,), 0)
    perms = [jnp.bitwise_and(iota + sh, 15) for sh in (8, 4, 2, 1)]
    x8 = jnp.bitwise_xor(iota, 8)
    x4 = jnp.bitwise_xor(iota, 4)
    x2 = jnp.bitwise_xor(iota, 2)
    x1 = jnp.bitwise_xor(iota, 1)
    m8 = jnp.bitwise_and(iota, 8) == 0
    m4 = jnp.bitwise_and(iota, 4) == 0
    m2 = jnp.bitwise_and(iota, 2) == 0
    lo8 = iota < 8
    peven = jnp.bitwise_and(iota * 2, 15)

    def _lanesum(acc):
        for perm in perms:
            acc = acc + jnp.take(acc, perm)
        return acc

    def _butterfly8(a):
        b = [jnp.where(m8, a[k], a[k + 4])
             + jnp.where(m8, jnp.take(a[k], x8), jnp.take(a[k + 4], x8))
             for k in range(4)]
        c = [jnp.where(m4, b[k], b[k + 2])
             + jnp.where(m4, jnp.take(b[k], x4), jnp.take(b[k + 2], x4))
             for k in range(2)]
        d = (jnp.where(m2, c[0], c[1])
             + jnp.where(m2, jnp.take(c[0], x2), jnp.take(c[1], x2)))
        return d + jnp.take(d, x1)

    for j in range(NCH):
        nbuf = nbufs[j % 4]
        pending[j].wait()

        def row_sums(m, nbuf=nbuf, j=j):
            bl = j * 16 + m
            teps = [tbuf[bl, pl.ds(c * 16, 16)] + EPS for c in range(8)]
            acc = jnp.zeros((16,), jnp.float32)
            for c in range(8):
                dv = (teps[c] - hbuf[bl, pl.ds(c * 16, 16)]
                      - rbuf[bl, pl.ds(c * 16, 16)])
                acc = acc + dv * dv
            accs = []
            for k in range(NEG):
                rr = m * NEG + k
                acc2 = jnp.zeros((16,), jnp.float32)
                for c in range(8):
                    dv = teps[c] - nbuf[rr, pl.ds(c * 16, 16)]
                    acc2 = acc2 + dv * dv
                accs.append(acc2)
            return _lanesum(acc), jnp.take(_butterfly8(accs), peven)

        def per_pair(i, pos_pack, j=j):
            m0 = 2 * i
            p0, w0 = row_sums(m0)
            p1, w1 = row_sums(m0 + 1)
            pos_pack = jnp.where(iota == m0, p0, pos_pack)
            pos_pack = jnp.where(iota == m0 + 1, p1, pos_pack)
            negsq8_v[pl.ds(j * 128 + i * 16, 16)] = jnp.where(lo8, w0, w1)
            return pos_pack

        zero16 = jnp.zeros((16,), jnp.float32)
        pos_pack = lax.fori_loop(0, 8, per_pair, zero16)
        possq_v[pl.ds(j * 16, 16)] = pos_pack
        if j + 4 < NCH:
            pending.append(pltpu.async_copy(
                ent_hbm.at[nidx_v.at[pl.ds((j + 4) * 128, 128)]],
                nbufs[j % 4], nsems[j % 4]))

    pltpu.sync_copy(negsq8_v, negsq_hbm.at[pl.ds(base * NEG, BPW * NEG)])
    pltpu.sync_copy(possq_v, pos32_hbm.at[wid])


@functools.partial(
    pl.kernel,
    out_type=(jax.ShapeDtypeStruct((B * NEG,), jnp.float32),
              jax.ShapeDtypeStruct((NW, BPW), jnp.float32)),
    mesh=plsc.VectorSubcoreMesh(core_axis_name="c", subcore_axis_name="s"),
    scratch_types=[
        pltpu.VMEM((BPW,), jnp.int32),
        pltpu.VMEM((BPW,), jnp.int32),
        pltpu.VMEM((BPW,), jnp.int32),
        pltpu.VMEM((BPW * NEG,), jnp.int32),
        pltpu.VMEM((BPW, D), jnp.float32),
        pltpu.VMEM((BPW, D), jnp.float32),
        pltpu.VMEM((BPW, D), jnp.float32),
        pltpu.VMEM((128, D), jnp.float32),
        pltpu.VMEM((128, D), jnp.float32),
        pltpu.VMEM((128, D), jnp.float32),
        pltpu.VMEM((128, D), jnp.float32),
        pltpu.VMEM((BPW,), jnp.float32),
        pltpu.VMEM((BPW * NEG,), jnp.float32),
        pltpu.SemaphoreType.DMA,
        pltpu.SemaphoreType.DMA,
        pltpu.SemaphoreType.DMA,
        pltpu.SemaphoreType.DMA,
        pltpu.SemaphoreType.DMA,
        pltpu.SemaphoreType.DMA,
        pltpu.SemaphoreType.DMA,
    ],
)
def _sc_call(ent_hbm, rel_hbm, idx_hbm, negsq_hbm, pos32_hbm, *scratch):
    _sc_body(ent_hbm, rel_hbm, idx_hbm, negsq_hbm, pos32_hbm, *scratch)


def _tc_body(negsq_ref, pos32_ref, out_ref):
    pos = jnp.sqrt(pos32_ref[:, :])
    negd = jnp.sqrt(negsq_ref[:, :])
    pos_sum = jnp.sum(pos)
    neg_sum = jnp.sum(negd) / NEG
    out_ref[:, :] = jnp.broadcast_to((pos_sum - neg_sum) / B + MARGIN, (1, 1))

    pos_min = jnp.min(pos)
    negd_max = jnp.max(negd)

    @pl.when(pos_min + MARGIN < negd_max)
    def _brute():
        li = lax.broadcasted_iota(jnp.int32, (D, 16), 0) // NEG
        gi = lax.broadcasted_iota(jnp.int32, (D, 16), 1)
        sel = jnp.where(li == gi, jnp.float32(1.0 / NEG), jnp.float32(0.0))
        nm16 = jax.lax.dot(negd, sel,
                           precision=jax.lax.Precision.HIGHEST)

        acc = jnp.zeros((256, D), jnp.float32)
        for rp in range(NW):
            prow = pos[rp:rp + 1, :]
            for g in range(16):
                a = nm16[:, g:g + 1]
                acc = acc + jnp.maximum(prow - a + MARGIN, 0.0)
        out_ref[:, :] = jnp.broadcast_to(jnp.sum(acc) / (B * B), (1, 1))


def _tc_call(negsq, pos32):
    return pl.pallas_call(
        _tc_body,
        out_shape=jax.ShapeDtypeStruct((1, 1), jnp.float32),
    )(negsq, pos32)


def kernel(sample, neg_idx, entity_table, rel_table):
    s32 = sample.astype(jnp.int32)
    idx_all = jnp.concatenate(
        [s32[:, 0], s32[:, 1], s32[:, 2],
         neg_idx.astype(jnp.int32).reshape(B * NEG)])
    negsq, pos32 = _sc_call(entity_table, rel_table, idx_all)
    total = _tc_call(negsq.reshape(B * NEG // D, D), pos32)
    return total[0, 0]
